# baseline (speedup 1.0000x reference)
import math
import functools

import jax
import jax.numpy as jnp
from jax import lax
from jax.experimental import pallas as pl
from jax.experimental.pallas import tpu as pltpu
from jax.experimental.pallas import tpu_sc as plsc

H = 768
NH = 12
DH = 64
HALF = DH // 2
E = 16
DFF = 256
EPS = 1e-06
THETA = 10000.0
S = 2048

TSA = 256
TSQ = 512
TSC = 512
TM = 128
NT = 32
TP = NT * TM

SC_NC = 2
SC_NS = 16
SC_NW = SC_NC * SC_NS
TOK_W = S // SC_NW


def _dot_t(a, b):
    return lax.dot_general(a, b, (((1,), (1,)), ((), ())),
                           preferred_element_type=jnp.float32)


def _silu(x):
    return x * (1.0 / (1.0 + jnp.exp(-x)))


def _stage_a_body(x_ref, w_in_ref, qkvw_ref, qln_ref, kln_ref,
                  q_ref, k_ref, v_ref):
    i = pl.program_id(0)
    x = x_ref[...]
    var = jnp.mean(x * x, axis=1, keepdims=True)
    xn = x * lax.rsqrt(var + EPS) * w_in_ref[...]
    qkv = _dot_t(xn, qkvw_ref[...])

    pos = (i * TSA + lax.broadcasted_iota(jnp.int32, (TSA, 1), 0)
           ).astype(jnp.float32)
    j = lax.broadcasted_iota(jnp.int32, (1, HALF), 1).astype(jnp.float32)
    inv_freq = jnp.exp(j * (-math.log(THETA) / HALF))
    ang = pos * inv_freq
    c = jnp.cos(ang)
    s = jnp.sin(ang)

    def norm_rope(mat, w):
        pieces = []
        for h in range(NH):
            xh = mat[:, h * DH:(h + 1) * DH]
            v_ = jnp.mean(xh * xh, axis=1, keepdims=True)
            xh = xh * lax.rsqrt(v_ + EPS) * w
            x1 = xh[:, :HALF]
            x2 = xh[:, HALF:]
            pieces.append(jnp.concatenate([x1 * c - x2 * s,
                                           x1 * s + x2 * c], axis=1))
        return jnp.concatenate(pieces, axis=1)

    q_ref[...] = norm_rope(qkv[:, :H], qln_ref[...])
    k_ref[...] = norm_rope(qkv[:, H:2 * H], kln_ref[...])
    v_ref[...] = qkv[:, 2 * H:]


def _stage_a(x, w_in, qkv_w, q_ln, k_ln):
    n = S // TSA
    return pl.pallas_call(
        _stage_a_body,
        grid=(n,),
        in_specs=[
            pl.BlockSpec((TSA, H), lambda i: (i, 0)),
            pl.BlockSpec((1, H), lambda i: (0, 0)),
            pl.BlockSpec((3 * H, H), lambda i: (0, 0)),
            pl.BlockSpec((1, DH), lambda i: (0, 0)),
            pl.BlockSpec((1, DH), lambda i: (0, 0)),
        ],
        out_specs=[
            pl.BlockSpec((TSA, H), lambda i: (i, 0)),
            pl.BlockSpec((TSA, H), lambda i: (i, 0)),
            pl.BlockSpec((TSA, H), lambda i: (i, 0)),
        ],
        out_shape=[jax.ShapeDtypeStruct((S, H), jnp.float32)] * 3,
    )(x, w_in, qkv_w, q_ln, k_ln)


KC = 512
NK = S // KC


def _stage_b_body(q_ref, k_ref, v_ref, o_ref, acc_ref, m0_ref, l0_ref,
                  m1_ref, l1_ref):
    qi = pl.program_id(1)
    ki = pl.program_id(2)

    @pl.when(ki == 0)
    def _():
        acc_ref[...] = jnp.zeros_like(acc_ref)
        for r in (m0_ref, m1_ref):
            r[...] = jnp.full_like(r, -3e38)
        for r in (l0_ref, l1_ref):
            r[...] = jnp.zeros_like(r)

    def update(masked):
        q = q_ref[...]
        k = k_ref[...]
        v = v_ref[...]
        if masked:
            rows = qi * TSQ + lax.broadcasted_iota(jnp.int32, (TSQ, 1), 0)
            cols = ki * KC + lax.broadcasted_iota(jnp.int32, (1, KC), 1)
            mask = cols <= rows
        for sub, (m_ref, l_ref) in enumerate(((m0_ref, l0_ref),
                                              (m1_ref, l1_ref))):
            qh = q[:, sub * DH:(sub + 1) * DH]
            kh = k[:, sub * DH:(sub + 1) * DH]
            vh = v[:, sub * DH:(sub + 1) * DH]
            sc = _dot_t(qh, kh) * (DH ** -0.5)
            if masked:
                sc = jnp.where(mask, sc, -1e9)
            m_old = m_ref[...]
            m_new = jnp.maximum(jnp.max(sc, axis=1, keepdims=True), m_old)
            alpha = jnp.exp(m_old - m_new)
            p = jnp.exp(sc - m_new)
            m_ref[...] = m_new
            l_ref[...] = l_ref[...] * alpha + jnp.sum(p, axis=1,
                                                      keepdims=True)
            pv = lax.dot_general(p, vh, (((1,), (0,)), ((), ())),
                                 preferred_element_type=jnp.float32)
            acc_ref[:, sub * DH:(sub + 1) * DH] = (
                acc_ref[:, sub * DH:(sub + 1) * DH] * alpha + pv)

    @pl.when(ki * KC < qi * TSQ)
    def _():
        update(False)

    @pl.when((ki * KC >= qi * TSQ) & (ki * KC < qi * TSQ + TSQ))
    def _():
        update(True)

    @pl.when(ki == NK - 1)
    def _():
        acc = acc_ref[...]
        o_ref[...] = jnp.concatenate(
            [acc[:, :DH] * (1.0 / l0_ref[...]),
             acc[:, DH:] * (1.0 / l1_ref[...])], axis=1)


def _stage_b(q, k, v):
    npair = NH // 2
    nq = S // TSQ
    return pl.pallas_call(
        _stage_b_body,
        grid=(npair, nq, NK),
        in_specs=[
            pl.BlockSpec((TSQ, 2 * DH), lambda p, qi, ki: (qi, p)),
            pl.BlockSpec((KC, 2 * DH), lambda p, qi, ki: (ki, p)),
            pl.BlockSpec((KC, 2 * DH), lambda p, qi, ki: (ki, p)),
        ],
        out_specs=pl.BlockSpec((TSQ, 2 * DH), lambda p, qi, ki: (qi, p)),
        out_shape=jax.ShapeDtypeStruct((S, H), jnp.float32),
        scratch_shapes=[
            pltpu.VMEM((TSQ, 2 * DH), jnp.float32),
            pltpu.VMEM((TSQ, 1), jnp.float32),
            pltpu.VMEM((TSQ, 1), jnp.float32),
            pltpu.VMEM((TSQ, 1), jnp.float32),
            pltpu.VMEM((TSQ, 1), jnp.float32),
        ],
    )(q, k, v)


def _stage_c_body(ctx_ref, hid_ref, ow_ref, postw_ref, routw_ref,
                  sg_ref, su_ref, sd_ref,
                  base_ref, h2_ref, topw_ref, ti_ref, rank_ref, cnt_ref):
    i = pl.program_id(0)
    ctx = ctx_ref[...]
    attn_out = _dot_t(ctx, ow_ref[...])
    hidden2 = hid_ref[...] + attn_out
    var = jnp.mean(hidden2 * hidden2, axis=1, keepdims=True)
    h2 = hidden2 * lax.rsqrt(var + EPS) * postw_ref[...]
    h2_ref[...] = h2

    logits = _dot_t(h2, routw_ref[...])
    m = jnp.max(logits, axis=1, keepdims=True)
    p = jnp.exp(logits - m)
    p = p / jnp.sum(p, axis=1, keepdims=True)
    topw = jnp.max(p, axis=1, keepdims=True)
    ie = lax.broadcasted_iota(jnp.int32, p.shape, 1)
    ti = jnp.min(jnp.where(p == topw, ie, E), axis=1, keepdims=True)
    topw_ref[...] = topw
    ti_ref[...] = ti

    @pl.when(i == 0)
    def _():
        cnt_ref[...] = jnp.zeros_like(cnt_ref)

    oh = (ie == ti).astype(jnp.float32)
    ir = lax.broadcasted_iota(jnp.int32, (TSC, TSC), 0)
    ic = lax.broadcasted_iota(jnp.int32, (TSC, TSC), 1)
    strict_lower = (ic < ir).astype(jnp.float32)
    excl = lax.dot_general(strict_lower, oh, (((1,), (0,)), ((), ())),
                           preferred_element_type=jnp.float32)
    base_cnt = cnt_ref[...]
    rank_ref[...] = jnp.sum(oh * (excl + base_cnt), axis=1, keepdims=True)
    cnt_ref[...] = base_cnt + jnp.sum(oh, axis=0, keepdims=True)

    g = _dot_t(h2, sg_ref[...])
    u = _dot_t(h2, su_ref[...])
    shared = _dot_t(_silu(g) * u, sd_ref[...])
    base_ref[...] = hidden2 + shared


def _stage_c(ctx, hidden, o_w, post_w, rout_w, sg, su, sd):
    n = S // TSC
    return pl.pallas_call(
        _stage_c_body,
        grid=(n,),
        in_specs=[
            pl.BlockSpec((TSC, H), lambda i: (i, 0)),
            pl.BlockSpec((TSC, H), lambda i: (i, 0)),
            pl.BlockSpec((H, H), lambda i: (0, 0)),
            pl.BlockSpec((1, H), lambda i: (0, 0)),
            pl.BlockSpec((E, H), lambda i: (0, 0)),
            pl.BlockSpec((DFF, H), lambda i: (0, 0)),
            pl.BlockSpec((DFF, H), lambda i: (0, 0)),
            pl.BlockSpec((H, DFF), lambda i: (0, 0)),
        ],
        out_specs=[
            pl.BlockSpec((TSC, H), lambda i: (i, 0)),
            pl.BlockSpec((TSC, H), lambda i: (i, 0)),
            pl.BlockSpec((TSC, 1), lambda i: (i, 0)),
            pl.BlockSpec((TSC, 1), lambda i: (i, 0)),
            pl.BlockSpec((TSC, 1), lambda i: (i, 0)),
            pl.BlockSpec((1, E), lambda i: (0, 0)),
        ],
        out_shape=[
            jax.ShapeDtypeStruct((S, H), jnp.float32),
            jax.ShapeDtypeStruct((S, H), jnp.float32),
            jax.ShapeDtypeStruct((S, 1), jnp.float32),
            jax.ShapeDtypeStruct((S, 1), jnp.int32),
            jax.ShapeDtypeStruct((S, 1), jnp.float32),
            jax.ShapeDtypeStruct((1, E), jnp.float32),
        ],
    )(ctx, hidden, o_w, post_w, rout_w, sg, su, sd)


def _stage_f_body(ti_ref, rank_ref, cnt_ref, dest_ref, eot_ref):
    cnt = cnt_ref[...]
    padded = jnp.floor((cnt + (TM - 1)) * (1.0 / TM)).astype(jnp.float32) * TM
    iel = lax.broadcasted_iota(jnp.int32, (E, E), 0)
    iec = lax.broadcasted_iota(jnp.int32, (E, E), 1)
    strict_lower = (iel < iec).astype(jnp.float32)
    off = lax.dot_general(padded, strict_lower, (((1,), (0,)), ((), ())),
                          preferred_element_type=jnp.float32)

    ti = ti_ref[...]
    ie = lax.broadcasted_iota(jnp.int32, (S, E), 1)
    oh = (ie == ti).astype(jnp.float32)
    dest = jnp.sum(oh * off, axis=1, keepdims=True) + rank_ref[...]
    dest_ref[...] = dest.astype(jnp.int32)

    end = off + padded
    tstart = (TM * lax.broadcasted_iota(jnp.int32, (1, NT), 1)
              ).astype(jnp.float32)
    ge = (jnp.broadcast_to(tstart.reshape(1, NT), (E, NT))
          >= jnp.broadcast_to(end.reshape(E, 1), (E, NT)))
    eot = jnp.sum(ge.astype(jnp.int32), axis=0, keepdims=True)
    eot_ref[...] = jnp.minimum(eot, E - 1)


def _stage_f(ti, rank, cnt):
    return pl.pallas_call(
        _stage_f_body,
        grid=(1,),
        in_specs=[
            pl.BlockSpec((S, 1), lambda i: (0, 0)),
            pl.BlockSpec((S, 1), lambda i: (0, 0)),
            pl.BlockSpec((1, E), lambda i: (0, 0)),
        ],
        out_specs=[
            pl.BlockSpec((S, 1), lambda i: (0, 0)),
            pl.BlockSpec((1, NT), lambda i: (0, 0)),
        ],
        out_shape=[
            jax.ShapeDtypeStruct((S, 1), jnp.int32),
            jax.ShapeDtypeStruct((1, NT), jnp.int32),
        ],
    )(ti, rank, cnt)


def _sc_mesh():
    return plsc.VectorSubcoreMesh(core_axis_name="c", subcore_axis_name="s",
                                  num_cores=SC_NC, num_subcores=SC_NS)


def _sc_scatter(h2, dest32):
    @functools.partial(
        pl.kernel,
        out_type=jax.ShapeDtypeStruct((TP, H), jnp.float32),
        mesh=_sc_mesh(),
        scratch_types=[pltpu.VMEM((TOK_W,), jnp.int32),
                       pltpu.VMEM((TOK_W, H), jnp.float32),
                       pltpu.SemaphoreType.DMA],
    )
    def k(h2_hbm, dest_hbm, out_hbm, idx_v, rows_v, sem):
        wid = lax.axis_index("s") * SC_NC + lax.axis_index("c")
        base = wid * TOK_W
        pltpu.sync_copy(dest_hbm.at[wid], idx_v)
        pltpu.sync_copy(h2_hbm.at[pl.ds(base, TOK_W)], rows_v)
        pltpu.async_copy(rows_v, out_hbm.at[idx_v], sem).wait()

    return k(h2, dest32)


def _sc_gather(sorted_rows, dest32):
    @functools.partial(
        pl.kernel,
        out_type=jax.ShapeDtypeStruct((S, H), jnp.float32),
        mesh=_sc_mesh(),
        scratch_types=[pltpu.VMEM((TOK_W,), jnp.int32),
                       pltpu.VMEM((TOK_W, H), jnp.float32),
                       pltpu.SemaphoreType.DMA],
    )
    def k(src_hbm, dest_hbm, out_hbm, idx_v, rows_v, sem):
        wid = lax.axis_index("s") * SC_NC + lax.axis_index("c")
        base = wid * TOK_W
        pltpu.sync_copy(dest_hbm.at[wid], idx_v)
        pltpu.async_copy(src_hbm.at[idx_v], rows_v, sem).wait()
        pltpu.sync_copy(rows_v, out_hbm.at[pl.ds(base, TOK_W)])

    return k(sorted_rows, dest32)


def _stage_d_body(eot_ref, hs_ref, gu_ref, dw_ref, o_ref):
    h = hs_ref[...]
    gup = _dot_t(h, gu_ref[0])
    g = gup[:, :DFF]
    u = gup[:, DFF:]
    inter = _silu(g) * u
    o_ref[...] = _dot_t(inter, dw_ref[0])


def _stage_d(eot, sorted_rows, gu_w, d_w):
    grid_spec = pltpu.PrefetchScalarGridSpec(
        num_scalar_prefetch=1,
        grid=(NT,),
        in_specs=[
            pl.BlockSpec((TM, H), lambda i, eot: (i, 0)),
            pl.BlockSpec((1, 2 * DFF, H), lambda i, eot: (eot[i], 0, 0)),
            pl.BlockSpec((1, H, DFF), lambda i, eot: (eot[i], 0, 0)),
        ],
        out_specs=pl.BlockSpec((TM, H), lambda i, eot: (i, 0)),
    )
    return pl.pallas_call(
        _stage_d_body,
        grid_spec=grid_spec,
        out_shape=jax.ShapeDtypeStruct((TP, H), jnp.float32),
    )(eot, sorted_rows, gu_w, d_w)


def _stage_e_body(base_ref, topw_ref, g_ref, o_ref):
    o_ref[...] = base_ref[...] + topw_ref[...] * g_ref[...]


def _stage_e(base, topw, gathered):
    n = S // TSC
    return pl.pallas_call(
        _stage_e_body,
        grid=(n,),
        in_specs=[
            pl.BlockSpec((TSC, H), lambda i: (i, 0)),
            pl.BlockSpec((TSC, 1), lambda i: (i, 0)),
            pl.BlockSpec((TSC, H), lambda i: (i, 0)),
        ],
        out_specs=pl.BlockSpec((TSC, H), lambda i: (i, 0)),
        out_shape=jax.ShapeDtypeStruct((S, H), jnp.float32),
    )(base, topw, gathered)


def kernel(hidden_states, input_ln_w, qkv_w, q_ln_w, k_ln_w, o_w, post_ln_w,
           router_w, expert_gate_up_w, expert_down_w, shared_gate_w,
           shared_up_w, shared_down_w):
    B = hidden_states.shape[0]
    x = hidden_states.reshape(S, H)
    q, k, v = _stage_a(x, input_ln_w.reshape(1, H), qkv_w,
                       q_ln_w.reshape(1, DH), k_ln_w.reshape(1, DH))
    ctx = _stage_b(q, k, v)
    base, h2, topw, ti, rank, cnt = _stage_c(
        ctx, x, o_w, post_ln_w.reshape(1, H), router_w,
        shared_gate_w, shared_up_w, shared_down_w)
    dest, eot = _stage_f(ti, rank, cnt)
    dest32 = dest.reshape(SC_NW, TOK_W)
    sorted_rows = _sc_scatter(h2, dest32)
    moe_sorted = _stage_d(eot.reshape(NT), sorted_rows,
                          expert_gate_up_w, expert_down_w)
    moe_g = _sc_gather(moe_sorted, dest32)
    out = _stage_e(base, topw, moe_g)
    return out.reshape(B, S, H)


# speedup vs baseline: 1.5230x; 1.5230x over previous
import math
import functools

import jax
import jax.numpy as jnp
from jax import lax
from jax.experimental import pallas as pl
from jax.experimental.pallas import tpu as pltpu
from jax.experimental.pallas import tpu_sc as plsc

H = 768
NH = 12
DH = 64
HALF = DH // 2
E = 16
DFF = 256
EPS = 1e-06
THETA = 10000.0
S = 2048

TSA = 256
TSQ = 512
TSC = 512
TM = 128
NT = 32
TP = NT * TM

SC_NC = 2
SC_NS = 16
SC_NW = SC_NC * SC_NS
TOK_W = S // SC_NW


def _dot_t(a, b):
    return lax.dot_general(a, b, (((1,), (1,)), ((), ())),
                           preferred_element_type=jnp.float32)


def _silu(x):
    return x * (1.0 / (1.0 + jnp.exp(-x)))


def _stage_a_body(x_ref, w_in_ref, qkvw_ref, bd_ref, qlnf_ref, klnf_ref,
                  q_ref, k_ref, v_ref):
    i = pl.program_id(0)
    x = x_ref[...]
    var = jnp.mean(x * x, axis=1, keepdims=True)
    xn = x * lax.rsqrt(var + EPS) * w_in_ref[...]
    qkv = _dot_t(xn, qkvw_ref[...])

    pos = (i * TSA + lax.broadcasted_iota(jnp.int32, (TSA, 1), 0)
           ).astype(jnp.float32)
    col = lax.broadcasted_iota(jnp.int32, (1, H), 1)
    j = lax.rem(col, DH)
    jj = lax.rem(j, HALF).astype(jnp.float32)
    inv_freq = jnp.exp(jj * (-math.log(THETA) / HALF))
    ang = pos * inv_freq
    c = jnp.cos(ang)
    s = jnp.sin(ang)
    first_half = j < HALF
    s_signed = jnp.where(first_half, -s, s)

    def norm_rope(mat, wfull):
        gs = lax.dot_general(mat * mat, bd_ref[...],
                             (((1,), (0,)), ((), ())),
                             preferred_element_type=jnp.float32)
        mn = mat * lax.rsqrt(gs * (1.0 / DH) + EPS) * wfull
        swapped = jnp.where(
            first_half,
            jnp.roll(mn, -HALF, axis=1),
            jnp.roll(mn, HALF, axis=1))
        return mn * c + swapped * s_signed

    q_ref[...] = norm_rope(qkv[:, :H], qlnf_ref[...])
    k_ref[...] = norm_rope(qkv[:, H:2 * H], klnf_ref[...])
    v_ref[...] = qkv[:, 2 * H:]


def _stage_a(x, w_in, qkv_w, bd, q_lnf, k_lnf):
    n = S // TSA
    return pl.pallas_call(
        _stage_a_body,
        grid=(n,),
        in_specs=[
            pl.BlockSpec((TSA, H), lambda i: (i, 0)),
            pl.BlockSpec((1, H), lambda i: (0, 0)),
            pl.BlockSpec((3 * H, H), lambda i: (0, 0)),
            pl.BlockSpec((H, H), lambda i: (0, 0)),
            pl.BlockSpec((1, H), lambda i: (0, 0)),
            pl.BlockSpec((1, H), lambda i: (0, 0)),
        ],
        out_specs=[
            pl.BlockSpec((TSA, H), lambda i: (i, 0)),
            pl.BlockSpec((TSA, H), lambda i: (i, 0)),
            pl.BlockSpec((TSA, H), lambda i: (i, 0)),
        ],
        out_shape=[jax.ShapeDtypeStruct((S, H), jnp.float32)] * 3,
    )(x, w_in, qkv_w, bd, q_lnf, k_lnf)


def _dot_nt(a, b):
    return lax.dot_general(a, b, (((1,), (0,)), ((), ())),
                           preferred_element_type=jnp.float32)


def _stage_b_call(q, k, v, qi):
    w = (qi + 1) * TSQ
    wm = w - TSQ
    scale = DH ** -0.5

    def body(q_ref, k_ref, v_ref, o_ref):
        qv = q_ref[...]
        kv = k_ref[...]
        vv = v_ref[...]
        rows = lax.broadcasted_iota(jnp.int32, (TSQ, 1), 0)
        cols = lax.broadcasted_iota(jnp.int32, (1, TSQ), 1)
        dmask = cols <= rows
        outs = []
        for sub in range(2):
            qh = qv[:, sub * DH:(sub + 1) * DH]
            kh = kv[:, sub * DH:(sub + 1) * DH]
            vh = vv[:, sub * DH:(sub + 1) * DH]
            sd = _dot_t(qh, kh[wm:, :]) * scale
            sd = jnp.where(dmask, sd, -1e9)
            m = jnp.max(sd, axis=1, keepdims=True)
            if wm > 0:
                sm = _dot_t(qh, kh[:wm, :]) * scale
                m = jnp.maximum(jnp.max(sm, axis=1, keepdims=True), m)
                pm = jnp.exp(sm - m)
                pd = jnp.exp(sd - m)
                l = (jnp.sum(pm, axis=1, keepdims=True)
                     + jnp.sum(pd, axis=1, keepdims=True))
                ctx = _dot_nt(pm, vh[:wm, :]) + _dot_nt(pd, vh[wm:, :])
            else:
                pd = jnp.exp(sd - m)
                l = jnp.sum(pd, axis=1, keepdims=True)
                ctx = _dot_nt(pd, vh)
            outs.append(ctx * (1.0 / l))
        o_ref[...] = jnp.concatenate(outs, axis=1)

    npair = NH // 2
    return pl.pallas_call(
        body,
        grid=(npair,),
        in_specs=[
            pl.BlockSpec((TSQ, 2 * DH), lambda p: (qi, p)),
            pl.BlockSpec((w, 2 * DH), lambda p: (0, p)),
            pl.BlockSpec((w, 2 * DH), lambda p: (0, p)),
        ],
        out_specs=pl.BlockSpec((TSQ, 2 * DH), lambda p: (0, p)),
        out_shape=jax.ShapeDtypeStruct((TSQ, H), jnp.float32),
    )(q, k, v)


def _stage_b(q, k, v):
    parts = [_stage_b_call(q, k, v, qi) for qi in range(S // TSQ)]
    return jnp.concatenate(parts, axis=0)


def _stage_c_body(ctx_ref, hid_ref, ow_ref, postw_ref, routw_ref,
                  sg_ref, su_ref, sd_ref,
                  base_ref, h2_ref, topw_ref, ti_ref, rank_ref, cnt_ref):
    i = pl.program_id(0)
    ctx = ctx_ref[...]
    attn_out = _dot_t(ctx, ow_ref[...])
    hidden2 = hid_ref[...] + attn_out
    var = jnp.mean(hidden2 * hidden2, axis=1, keepdims=True)
    h2 = hidden2 * lax.rsqrt(var + EPS) * postw_ref[...]
    h2_ref[...] = h2

    logits = _dot_t(h2, routw_ref[...])
    m = jnp.max(logits, axis=1, keepdims=True)
    p = jnp.exp(logits - m)
    p = p / jnp.sum(p, axis=1, keepdims=True)
    topw = jnp.max(p, axis=1, keepdims=True)
    ie = lax.broadcasted_iota(jnp.int32, p.shape, 1)
    ti = jnp.min(jnp.where(p == topw, ie, E), axis=1, keepdims=True)
    topw_ref[...] = topw
    ti_ref[...] = ti

    @pl.when(i == 0)
    def _():
        cnt_ref[...] = jnp.zeros_like(cnt_ref)

    oh = (ie == ti).astype(jnp.float32)
    ir = lax.broadcasted_iota(jnp.int32, (TSC, TSC), 0)
    ic = lax.broadcasted_iota(jnp.int32, (TSC, TSC), 1)
    strict_lower = (ic < ir).astype(jnp.float32)
    excl = lax.dot_general(strict_lower, oh, (((1,), (0,)), ((), ())),
                           preferred_element_type=jnp.float32)
    base_cnt = cnt_ref[...]
    rank_ref[...] = jnp.sum(oh * (excl + base_cnt), axis=1, keepdims=True)
    cnt_ref[...] = base_cnt + jnp.sum(oh, axis=0, keepdims=True)

    g = _dot_t(h2, sg_ref[...])
    u = _dot_t(h2, su_ref[...])
    shared = _dot_t(_silu(g) * u, sd_ref[...])
    base_ref[...] = hidden2 + shared


def _stage_c(ctx, hidden, o_w, post_w, rout_w, sg, su, sd):
    n = S // TSC
    return pl.pallas_call(
        _stage_c_body,
        grid=(n,),
        in_specs=[
            pl.BlockSpec((TSC, H), lambda i: (i, 0)),
            pl.BlockSpec((TSC, H), lambda i: (i, 0)),
            pl.BlockSpec((H, H), lambda i: (0, 0)),
            pl.BlockSpec((1, H), lambda i: (0, 0)),
            pl.BlockSpec((E, H), lambda i: (0, 0)),
            pl.BlockSpec((DFF, H), lambda i: (0, 0)),
            pl.BlockSpec((DFF, H), lambda i: (0, 0)),
            pl.BlockSpec((H, DFF), lambda i: (0, 0)),
        ],
        out_specs=[
            pl.BlockSpec((TSC, H), lambda i: (i, 0)),
            pl.BlockSpec((TSC, H), lambda i: (i, 0)),
            pl.BlockSpec((TSC, 1), lambda i: (i, 0)),
            pl.BlockSpec((TSC, 1), lambda i: (i, 0)),
            pl.BlockSpec((TSC, 1), lambda i: (i, 0)),
            pl.BlockSpec((1, E), lambda i: (0, 0)),
        ],
        out_shape=[
            jax.ShapeDtypeStruct((S, H), jnp.float32),
            jax.ShapeDtypeStruct((S, H), jnp.float32),
            jax.ShapeDtypeStruct((S, 1), jnp.float32),
            jax.ShapeDtypeStruct((S, 1), jnp.int32),
            jax.ShapeDtypeStruct((S, 1), jnp.float32),
            jax.ShapeDtypeStruct((1, E), jnp.float32),
        ],
    )(ctx, hidden, o_w, post_w, rout_w, sg, su, sd)


def _stage_f_body(ti_ref, rank_ref, cnt_ref, dest_ref, eot_ref):
    cnt = cnt_ref[...]
    padded = jnp.floor((cnt + (TM - 1)) * (1.0 / TM)).astype(jnp.float32) * TM
    iel = lax.broadcasted_iota(jnp.int32, (E, E), 0)
    iec = lax.broadcasted_iota(jnp.int32, (E, E), 1)
    strict_lower = (iel < iec).astype(jnp.float32)
    off = lax.dot_general(padded, strict_lower, (((1,), (0,)), ((), ())),
                          preferred_element_type=jnp.float32)

    ti = ti_ref[...]
    ie = lax.broadcasted_iota(jnp.int32, (S, E), 1)
    oh = (ie == ti).astype(jnp.float32)
    dest = jnp.sum(oh * off, axis=1, keepdims=True) + rank_ref[...]
    dest_ref[...] = dest.astype(jnp.int32)

    end = off + padded
    tstart = (TM * lax.broadcasted_iota(jnp.int32, (1, NT), 1)
              ).astype(jnp.float32)
    ge = (jnp.broadcast_to(tstart.reshape(1, NT), (E, NT))
          >= jnp.broadcast_to(end.reshape(E, 1), (E, NT)))
    eot = jnp.sum(ge.astype(jnp.int32), axis=0, keepdims=True)
    eot_ref[...] = jnp.minimum(eot, E - 1)


def _stage_f(ti, rank, cnt):
    return pl.pallas_call(
        _stage_f_body,
        grid=(1,),
        in_specs=[
            pl.BlockSpec((S, 1), lambda i: (0, 0)),
            pl.BlockSpec((S, 1), lambda i: (0, 0)),
            pl.BlockSpec((1, E), lambda i: (0, 0)),
        ],
        out_specs=[
            pl.BlockSpec((S, 1), lambda i: (0, 0)),
            pl.BlockSpec((1, NT), lambda i: (0, 0)),
        ],
        out_shape=[
            jax.ShapeDtypeStruct((S, 1), jnp.int32),
            jax.ShapeDtypeStruct((1, NT), jnp.int32),
        ],
    )(ti, rank, cnt)


def _sc_mesh():
    return plsc.VectorSubcoreMesh(core_axis_name="c", subcore_axis_name="s",
                                  num_cores=SC_NC, num_subcores=SC_NS)


def _sc_scatter(h2, dest32):
    @functools.partial(
        pl.kernel,
        out_type=jax.ShapeDtypeStruct((TP, H), jnp.float32),
        mesh=_sc_mesh(),
        scratch_types=[pltpu.VMEM((TOK_W,), jnp.int32),
                       pltpu.VMEM((TOK_W, H), jnp.float32),
                       pltpu.SemaphoreType.DMA],
    )
    def k(h2_hbm, dest_hbm, out_hbm, idx_v, rows_v, sem):
        wid = lax.axis_index("s") * SC_NC + lax.axis_index("c")
        base = wid * TOK_W
        pltpu.sync_copy(dest_hbm.at[wid], idx_v)
        pltpu.sync_copy(h2_hbm.at[pl.ds(base, TOK_W)], rows_v)
        pltpu.async_copy(rows_v, out_hbm.at[idx_v], sem).wait()

    return k(h2, dest32)


def _sc_gather(sorted_rows, dest32):
    @functools.partial(
        pl.kernel,
        out_type=jax.ShapeDtypeStruct((S, H), jnp.float32),
        mesh=_sc_mesh(),
        scratch_types=[pltpu.VMEM((TOK_W,), jnp.int32),
                       pltpu.VMEM((TOK_W, H), jnp.float32),
                       pltpu.SemaphoreType.DMA],
    )
    def k(src_hbm, dest_hbm, out_hbm, idx_v, rows_v, sem):
        wid = lax.axis_index("s") * SC_NC + lax.axis_index("c")
        base = wid * TOK_W
        pltpu.sync_copy(dest_hbm.at[wid], idx_v)
        pltpu.async_copy(src_hbm.at[idx_v], rows_v, sem).wait()
        pltpu.sync_copy(rows_v, out_hbm.at[pl.ds(base, TOK_W)])

    return k(sorted_rows, dest32)


def _stage_d_body(eot_ref, hs_ref, gu_ref, dw_ref, o_ref):
    h = hs_ref[...]
    gup = _dot_t(h, gu_ref[0])
    g = gup[:, :DFF]
    u = gup[:, DFF:]
    inter = _silu(g) * u
    o_ref[...] = _dot_t(inter, dw_ref[0])


def _stage_d(eot, sorted_rows, gu_w, d_w):
    grid_spec = pltpu.PrefetchScalarGridSpec(
        num_scalar_prefetch=1,
        grid=(NT,),
        in_specs=[
            pl.BlockSpec((TM, H), lambda i, eot: (i, 0)),
            pl.BlockSpec((1, 2 * DFF, H), lambda i, eot: (eot[i], 0, 0)),
            pl.BlockSpec((1, H, DFF), lambda i, eot: (eot[i], 0, 0)),
        ],
        out_specs=pl.BlockSpec((TM, H), lambda i, eot: (i, 0)),
    )
    return pl.pallas_call(
        _stage_d_body,
        grid_spec=grid_spec,
        out_shape=jax.ShapeDtypeStruct((TP, H), jnp.float32),
    )(eot, sorted_rows, gu_w, d_w)


def _stage_e_body(base_ref, topw_ref, g_ref, o_ref):
    o_ref[...] = base_ref[...] + topw_ref[...] * g_ref[...]


def _stage_e(base, topw, gathered):
    n = S // TSC
    return pl.pallas_call(
        _stage_e_body,
        grid=(n,),
        in_specs=[
            pl.BlockSpec((TSC, H), lambda i: (i, 0)),
            pl.BlockSpec((TSC, 1), lambda i: (i, 0)),
            pl.BlockSpec((TSC, H), lambda i: (i, 0)),
        ],
        out_specs=pl.BlockSpec((TSC, H), lambda i: (i, 0)),
        out_shape=jax.ShapeDtypeStruct((S, H), jnp.float32),
    )(base, topw, gathered)


def kernel(hidden_states, input_ln_w, qkv_w, q_ln_w, k_ln_w, o_w, post_ln_w,
           router_w, expert_gate_up_w, expert_down_w, shared_gate_w,
           shared_up_w, shared_down_w):
    B = hidden_states.shape[0]
    x = hidden_states.reshape(S, H)
    hidx = jnp.arange(H) // DH
    bd = (hidx[:, None] == hidx[None, :]).astype(jnp.float32)
    q_lnf = jnp.tile(q_ln_w, NH).reshape(1, H)
    k_lnf = jnp.tile(k_ln_w, NH).reshape(1, H)
    q, k, v = _stage_a(x, input_ln_w.reshape(1, H), qkv_w, bd, q_lnf, k_lnf)
    ctx = _stage_b(q, k, v)
    base, h2, topw, ti, rank, cnt = _stage_c(
        ctx, x, o_w, post_ln_w.reshape(1, H), router_w,
        shared_gate_w, shared_up_w, shared_down_w)
    dest, eot = _stage_f(ti, rank, cnt)
    dest32 = dest.reshape(SC_NW, TOK_W)
    sorted_rows = _sc_scatter(h2, dest32)
    moe_sorted = _stage_d(eot.reshape(NT), sorted_rows,
                          expert_gate_up_w, expert_down_w)
    moe_g = _sc_gather(moe_sorted, dest32)
    out = _stage_e(base, topw, moe_g)
    return out.reshape(B, S, H)
